# baseline (device time: 65921 ns/iter reference)
import jax
import jax.numpy as jnp
from jax import lax
from jax.experimental import pallas as pl
from jax.experimental.pallas import tpu as pltpu

N_DEV = 16
BP = 128
B = N_DEV * BP
D = 128
HP = 256
GROUP = 4


def kernel(x, Win0, Wout0, Win1, Wout1, Win2, Wout2):
    def body(x_ref, win0, wout0, win1, wout1, win2, wout2, out_ref,
             xfull, p_stage, rs_buf, ag_send, ag_recv, rs_send, rs_recv):
        my = lax.axis_index("i")

        def blk(s):
            return pl.ds(s * BP, BP)

        def ag_send_all(bi):
            sends = []
            for off in range(1, N_DEV):
                dst = lax.rem(my + off, N_DEV)
                r = pltpu.make_async_remote_copy(
                    src_ref=xfull.at[bi, blk(0), :],
                    dst_ref=xfull.at[bi, blk(off), :],
                    send_sem=ag_send.at[bi, off],
                    recv_sem=ag_recv.at[bi, off],
                    device_id=(dst,),
                    device_id_type=pl.DeviceIdType.MESH,
                )
                r.start()
                sends.append(r)
            return sends

        def ag_wait_one(bi, s):
            pltpu.make_async_remote_copy(
                src_ref=xfull.at[bi, blk(0), :],
                dst_ref=xfull.at[bi, blk(s), :],
                send_sem=ag_send.at[bi, s],
                recv_sem=ag_recv.at[bi, s],
                device_id=(my,),
                device_id_type=pl.DeviceIdType.MESH,
            ).wait_recv()

        def layer(l, win, wout):
            bi = l % 2
            wi16 = win[...].astype(jnp.bfloat16)
            wo16 = wout[...].astype(jnp.bfloat16)
            rs_sends = []
            for g in range(N_DEV // GROUP):
                s0 = g * GROUP
                for s in range(max(s0, 1), s0 + GROUP):
                    ag_wait_one(bi, s)
                rows = pl.ds(s0 * BP, GROUP * BP)
                xc = xfull[bi, rows, :]
                h = jnp.dot(xc, wi16, preferred_element_type=jnp.float32)
                h = jnp.maximum(h, 0.0).astype(jnp.bfloat16)
                p = jnp.dot(h, wo16, preferred_element_type=jnp.float32)
                p_stage[pl.ds(s0, GROUP)] = (
                    p.astype(jnp.bfloat16).reshape(GROUP, BP, D)
                )
                for s in range(s0, s0 + GROUP):
                    if s == 0:
                        rs_buf[bi, 0] = p_stage[0]
                        continue
                    dst = lax.rem(my - s + N_DEV, N_DEV)
                    r = pltpu.make_async_remote_copy(
                        src_ref=p_stage.at[s],
                        dst_ref=rs_buf.at[bi, s],
                        send_sem=rs_send.at[bi, s],
                        recv_sem=rs_recv.at[bi, s],
                        device_id=(dst,),
                        device_id_type=pl.DeviceIdType.MESH,
                    )
                    r.start()
                    rs_sends.append(r)
            for s in range(1, N_DEV):
                pltpu.make_async_remote_copy(
                    src_ref=p_stage.at[0],
                    dst_ref=rs_buf.at[bi, s],
                    send_sem=rs_send.at[bi, s],
                    recv_sem=rs_recv.at[bi, s],
                    device_id=(my,),
                    device_id_type=pl.DeviceIdType.MESH,
                ).wait_recv()
            red = jnp.sum(rs_buf[bi].astype(jnp.float32), axis=0)

            nbi = (l + 1) % 2
            xfull[nbi, blk(0), :] = red.astype(jnp.bfloat16)
            ag_sends = ag_send_all(nbi)
            return ag_sends, rs_sends

        xfull[0, blk(0), :] = x_ref[...].astype(jnp.bfloat16)
        pending = ag_send_all(0)

        layers = [(win0, wout0), (win1, wout1), (win2, wout2)]
        for l, (wi, wo) in enumerate(layers):
            ag_sends, rs_sends = layer(l, wi, wo)
            for r in pending:
                r.wait_send()
            for r in rs_sends:
                r.wait_send()
            pending = ag_sends

        out_ref[pl.ds(my * BP, BP), :] = xfull[1, blk(0), :].astype(jnp.float32)
        for s in range(1, N_DEV):
            ag_wait_one(1, s)
            src_dev = lax.rem(my - s + N_DEV, N_DEV)
            out_ref[pl.ds(src_dev * BP, BP), :] = (
                xfull[1, blk(s), :].astype(jnp.float32)
            )
        for r in pending:
            r.wait_send()

    return pl.pallas_call(
        body,
        out_shape=jax.ShapeDtypeStruct((B, D), jnp.float32),
        in_specs=[pl.BlockSpec(memory_space=pltpu.VMEM)] * 7,
        out_specs=pl.BlockSpec(memory_space=pltpu.VMEM),
        scratch_shapes=[
            pltpu.VMEM((2, B, D), jnp.bfloat16),
            pltpu.VMEM((N_DEV, BP, D), jnp.bfloat16),
            pltpu.VMEM((2, N_DEV, BP, D), jnp.bfloat16),
            pltpu.SemaphoreType.DMA((2, N_DEV)),
            pltpu.SemaphoreType.DMA((2, N_DEV)),
            pltpu.SemaphoreType.DMA((2, N_DEV)),
            pltpu.SemaphoreType.DMA((2, N_DEV)),
        ],
    )(x, Win0, Wout0, Win1, Wout1, Win2, Wout2)
